# Initial kernel scaffold; baseline (speedup 1.0000x reference)
import jax
import jax.numpy as jnp
from jax.experimental import pallas as pl


def kernel(x_nchw, gamma, beta):
    raise NotImplementedError("write your pallas kernel here")


# speedup vs baseline: 1.0161x; 1.0161x over previous
import functools

import jax
import jax.numpy as jnp
from jax import lax
from jax.experimental import pallas as pl
from jax.experimental.pallas import tpu as pltpu

EPS = 1e-5
_VMEM_LIMIT = 48 << 20


def _stats_kernel(x_ref, sum_ref, sumsq_ref):
    @pl.when(pl.program_id(1) == 0)
    def _():
        sum_ref[...] = jnp.zeros_like(sum_ref)
        sumsq_ref[...] = jnp.zeros_like(sumsq_ref)

    x = x_ref[0]
    sum_ref[...] += jnp.sum(x, axis=-1, keepdims=True)
    sumsq_ref[...] += jnp.sum(x * x, axis=-1, keepdims=True)


def _norm_kernel(x_ref, sum_ref, sumsq_ref, gamma_ref, beta_ref, o_ref, *,
                 inv_count):
    mean = sum_ref[...] * inv_count
    var = jnp.maximum(sumsq_ref[...] * inv_count - mean * mean, 0.0)
    scale = gamma_ref[...] * lax.rsqrt(var + EPS)
    shift = beta_ref[...] - mean * scale
    o_ref[0] = x_ref[0] * scale + shift


@jax.jit
def _bn2d(x_nchw, gamma, beta):
    n, c, h, w = x_nchw.shape
    l = h * w
    x = x_nchw.reshape(n, c, l)
    c_tile = c if c <= 128 else 128
    cb = c // c_tile

    sum_o, sumsq_o = pl.pallas_call(
        _stats_kernel,
        out_shape=(jax.ShapeDtypeStruct((c, 1), jnp.float32),
                   jax.ShapeDtypeStruct((c, 1), jnp.float32)),
        grid=(cb, n),
        in_specs=[pl.BlockSpec((1, c_tile, l), lambda ci, ni: (ni, ci, 0))],
        out_specs=(pl.BlockSpec((c_tile, 1), lambda ci, ni: (ci, 0)),
                   pl.BlockSpec((c_tile, 1), lambda ci, ni: (ci, 0))),
        compiler_params=pltpu.CompilerParams(
            dimension_semantics=("parallel", "arbitrary"),
            vmem_limit_bytes=_VMEM_LIMIT),
    )(x)

    y = pl.pallas_call(
        functools.partial(_norm_kernel, inv_count=1.0 / float(n * l)),
        out_shape=jax.ShapeDtypeStruct((n, c, l), x_nchw.dtype),
        grid=(n, cb),
        in_specs=[pl.BlockSpec((1, c_tile, l), lambda ni, ci: (ni, ci, 0)),
                  pl.BlockSpec((c_tile, 1), lambda ni, ci: (ci, 0)),
                  pl.BlockSpec((c_tile, 1), lambda ni, ci: (ci, 0)),
                  pl.BlockSpec((c_tile, 1), lambda ni, ci: (ci, 0)),
                  pl.BlockSpec((c_tile, 1), lambda ni, ci: (ci, 0))],
        out_specs=pl.BlockSpec((1, c_tile, l), lambda ni, ci: (ni, ci, 0)),
        compiler_params=pltpu.CompilerParams(
            dimension_semantics=("parallel", "parallel"),
            vmem_limit_bytes=_VMEM_LIMIT),
    )(x, sum_o, sumsq_o,
      gamma.astype(jnp.float32).reshape(c, 1),
      beta.astype(jnp.float32).reshape(c, 1))

    return y.reshape(n, c, h, w)


def kernel(x_nchw, gamma, beta):
    return _bn2d(x_nchw, gamma, beta)
